# baseline (speedup 1.0000x reference)
import jax
import jax.numpy as jnp
from jax.experimental import pallas as pl
from jax.experimental.pallas import tpu as pltpu

BATCH = 1024
UNITS = 2048
D_IN = 512
CT = 128


def _cell_body(x_ref, ps_ref, k_ref, r_ref, b_ref, o_ref):
    ip = jnp.dot(x_ref[...], k_ref[...], preferred_element_type=jnp.float32)
    sp = jnp.dot(ps_ref[...], r_ref[...], preferred_element_type=jnp.float32)
    o_ref[...] = jnp.tanh(ip + sp + b_ref[...])


def kernel(inputs, prev_state, kernel, recurrent_kernel, bias):
    bias2 = bias.reshape(1, UNITS)
    out = pl.pallas_call(
        _cell_body,
        grid=(UNITS // CT,),
        in_specs=[
            pl.BlockSpec((BATCH, D_IN), lambda j: (0, 0)),
            pl.BlockSpec((BATCH, UNITS), lambda j: (0, 0)),
            pl.BlockSpec((D_IN, CT), lambda j: (0, j)),
            pl.BlockSpec((UNITS, CT), lambda j: (0, j)),
            pl.BlockSpec((1, CT), lambda j: (0, j)),
        ],
        out_specs=pl.BlockSpec((BATCH, CT), lambda j: (0, j)),
        out_shape=jax.ShapeDtypeStruct((BATCH, UNITS), jnp.float32),
    )(inputs, prev_state, kernel, recurrent_kernel, bias2)
    return out


# speedup vs baseline: 1.5375x; 1.5375x over previous
import jax
import jax.numpy as jnp
from jax.experimental import pallas as pl
from jax.experimental.pallas import tpu as pltpu

BATCH = 1024
UNITS = 2048
D_IN = 512
CT = 512


def _cell_body(x_ref, ps_ref, k_ref, r_ref, b_ref, o_ref):
    ip = jnp.dot(x_ref[...], k_ref[...], preferred_element_type=jnp.float32)
    sp = jnp.dot(ps_ref[...], r_ref[...], preferred_element_type=jnp.float32)
    o_ref[...] = jnp.tanh(ip + sp + b_ref[...])


def kernel(inputs, prev_state, kernel, recurrent_kernel, bias):
    bias2 = bias.reshape(1, UNITS)
    out = pl.pallas_call(
        _cell_body,
        grid=(UNITS // CT,),
        in_specs=[
            pl.BlockSpec((BATCH, D_IN), lambda j: (0, 0)),
            pl.BlockSpec((BATCH, UNITS), lambda j: (0, 0)),
            pl.BlockSpec((D_IN, CT), lambda j: (0, j)),
            pl.BlockSpec((UNITS, CT), lambda j: (0, j)),
            pl.BlockSpec((1, CT), lambda j: (0, j)),
        ],
        out_specs=pl.BlockSpec((BATCH, CT), lambda j: (0, j)),
        out_shape=jax.ShapeDtypeStruct((BATCH, UNITS), jnp.float32),
    )(inputs, prev_state, kernel, recurrent_kernel, bias2)
    return out
